# baseline (speedup 1.0000x reference)
import functools

import jax
import jax.numpy as jnp
from jax.experimental import pallas as pl
from jax.experimental.pallas import tpu as pltpu

F32 = jnp.float32
_H = 256


def _silu(v):
    return v * jax.nn.sigmoid(v)


def _premul_body(x_ref, w1a_ref, w1b_ref, b1_ref, wp_ref, bp_ref,
                 xa_ref, xb_ref, pp_ref):
    x = x_ref[...]
    xa_ref[...] = jnp.dot(x, w1a_ref[...], preferred_element_type=F32)
    xb_ref[...] = jnp.dot(x, w1b_ref[...], preferred_element_type=F32) + b1_ref[...]
    pp_ref[...] = jnp.dot(x, wp_ref[...], preferred_element_type=F32) + bp_ref[...]


def _premul(x, w1a, w1b, b1, wp, bp, nb=1000):
    n = x.shape[0]
    grid = (n // nb,)
    full = lambda r, c: pl.BlockSpec((r, c), lambda i: (0, 0))
    blk = lambda c: pl.BlockSpec((nb, c), lambda i: (i, 0))
    return pl.pallas_call(
        _premul_body,
        grid=grid,
        in_specs=[blk(_H), full(_H, _H), full(_H, _H), full(1, _H),
                  full(_H, 8), full(1, 8)],
        out_specs=[blk(_H), blk(_H), blk(8)],
        out_shape=[jax.ShapeDtypeStruct((n, _H), F32),
                   jax.ShapeDtypeStruct((n, _H), F32),
                   jax.ShapeDtypeStruct((n, 8), F32)],
    )(x, w1a, w1b, b1, wp, bp)


def _edge_body(h1pre_ref, geom_ref, w1c_ref, w2_ref, b2_ref, finf_ref,
               finfb_ref, wpc_ref, wp2_ref, bp2_ref, inv_n_ref,
               msg_ref, pmsg_ref):
    d2 = geom_ref[:, 0:1]
    dist = jnp.sqrt(d2)
    h1 = _silu(h1pre_ref[...] + dist * w1c_ref[...])
    m = _silu(jnp.dot(h1, w2_ref[...], preferred_element_type=F32) + b2_ref[...])
    logit = jnp.sum(m * finf_ref[...], axis=1, keepdims=True) + finfb_ref[...]
    msg_ref[...] = jax.nn.sigmoid(logit) * m
    ps = geom_ref[:, 1:4]
    p1 = _silu(ps + dist * wpc_ref[0:1, 0:3])
    mpos = (p1[:, 0:1] * wp2_ref[0:1, 0:3]
            + p1[:, 1:2] * wp2_ref[1:2, 0:3]
            + p1[:, 2:3] * wp2_ref[2:3, 0:3]) + bp2_ref[0:1, 0:3]
    mpos = _silu(mpos)
    sc = dist * inv_n_ref[0, 0]
    pmsg_ref[...] = jnp.concatenate(
        [sc * mpos, jnp.zeros((mpos.shape[0], 5), F32)], axis=1)


def _edge_mlp(h1pre, geom, w1c, w2, b2, finf_row, finf_b, wpc, wp2, bp2,
              inv_n, eb=1000):
    e = h1pre.shape[0]
    grid = (e // eb,)
    full = lambda r, c: pl.BlockSpec((r, c), lambda i: (0, 0))
    blk = lambda c: pl.BlockSpec((eb, c), lambda i: (i, 0))
    return pl.pallas_call(
        _edge_body,
        grid=grid,
        in_specs=[blk(_H), blk(8), full(1, _H), full(_H, _H), full(1, _H),
                  full(1, _H), full(1, 1), full(1, 8), full(8, 8),
                  full(1, 8), full(1, 1)],
        out_specs=[blk(_H), blk(8)],
        out_shape=[jax.ShapeDtypeStruct((e, _H), F32),
                   jax.ShapeDtypeStruct((e, 8), F32)],
    )(h1pre, geom, w1c, w2, b2, finf_row, finf_b, wpc, wp2, bp2, inv_n)


def _node_body(x_ref, mi_ref, pos_ref, mip_ref, w1a_ref, w1b_ref, b1_ref,
               w2_ref, b2_ref, pw1_ref, pb1_ref, pw2_ref, pb2_ref,
               xn_ref, pn_ref):
    x = x_ref[...]
    h = (jnp.dot(x, w1a_ref[...], preferred_element_type=F32)
         + jnp.dot(mi_ref[...], w1b_ref[...], preferred_element_type=F32)
         + b1_ref[...])
    h = _silu(h)
    xn_ref[...] = x + jnp.dot(h, w2_ref[...], preferred_element_type=F32) + b2_ref[...]
    pos3 = pos_ref[:, 0:3]
    mip3 = mip_ref[:, 0:3]
    hp = pb1_ref[0:1, 0:3]
    for k in range(3):
        hp = hp + pos3[:, k:k + 1] * pw1_ref[k:k + 1, 0:3]
        hp = hp + mip3[:, k:k + 1] * pw1_ref[k + 3:k + 4, 0:3]
    p1 = _silu(hp)
    pd = (p1[:, 0:1] * pw2_ref[0:1, 0:3]
          + p1[:, 1:2] * pw2_ref[1:2, 0:3]
          + p1[:, 2:3] * pw2_ref[2:3, 0:3]) + pb2_ref[0:1, 0:3]
    pn_ref[...] = jnp.concatenate(
        [pos3 + pd, jnp.zeros((pd.shape[0], 5), F32)], axis=1)


def _node_mlp(x, mi, pos8, mip8, w1a, w1b, b1, w2, b2, pw1, pb1, pw2, pb2,
              nb=1000):
    n = x.shape[0]
    grid = (n // nb,)
    full = lambda r, c: pl.BlockSpec((r, c), lambda i: (0, 0))
    blk = lambda c: pl.BlockSpec((nb, c), lambda i: (i, 0))
    return pl.pallas_call(
        _node_body,
        grid=grid,
        in_specs=[blk(_H), blk(_H), blk(8), blk(8), full(_H, _H),
                  full(_H, _H), full(1, _H), full(_H, _H), full(1, _H),
                  full(8, 8), full(1, 8), full(8, 8), full(1, 8)],
        out_specs=[blk(_H), blk(8)],
        out_shape=[jax.ShapeDtypeStruct((n, _H), F32),
                   jax.ShapeDtypeStruct((n, 8), F32)],
    )(x, mi, pos8, mip8, w1a, w1b, b1, w2, b2, pw1, pb1, pw2, pb2)


def kernel(x, edge_index, pos,
           fe_w1, fe_b1, fe_w2, fe_b2,
           finf_w, finf_b,
           fh_w1, fh_b1, fh_w2, fh_b2,
           fpos_w1, fpos_b1, fpos_w2, fpos_b2,
           fhpos_w1, fhpos_b1, fhpos_w2, fhpos_b2):
    n = x.shape[0]
    e = edge_index.shape[0]
    e_st = edge_index[:, 0]
    e_end = edge_index[:, 1]

    w1a = fe_w1[:_H]
    w1b = fe_w1[_H:2 * _H]
    w1c = fe_w1[2 * _H:2 * _H + 1]
    b1 = fe_b1.reshape(1, _H)
    wpa = fpos_w1[:_H]
    wpb = fpos_w1[_H:2 * _H]
    wpc = jnp.pad(fpos_w1[2 * _H:2 * _H + 1], ((0, 0), (0, 5)))
    wp = jnp.pad(jnp.concatenate([wpa, wpb], axis=1), ((0, 0), (0, 2)))
    bp = jnp.pad(jnp.concatenate(
        [jnp.zeros((3,), F32), fpos_b1]).reshape(1, 6), ((0, 0), (0, 2)))
    wp2 = jnp.pad(fpos_w2, ((0, 5), (0, 5)))
    bp2 = jnp.pad(fpos_b2.reshape(1, 3), ((0, 0), (0, 5)))
    finf_row = finf_w.reshape(1, _H)
    finfb = finf_b.reshape(1, 1)
    inv_n = jnp.full((1, 1), 1.0 / n, F32)

    xa, xb, pp = _premul(x, w1a, w1b, b1, wp, bp)
    pa = pp[:, 0:3]
    pb = pp[:, 3:6]

    h1pre = xa[e_st] + xb[e_end]
    diff = pos[e_st] - pos[e_end]
    d2 = jnp.sum(diff * diff, axis=1, keepdims=True)
    ps = pa[e_st] + pb[e_end]
    geom = jnp.concatenate([d2, ps, jnp.zeros((e, 4), F32)], axis=1)

    msg, pmsg = _edge_mlp(h1pre, geom, w1c, fe_w2, fe_b2.reshape(1, _H),
                          finf_row, finfb, wpc, wp2, bp2, inv_n)

    m_i = jax.ops.segment_sum(msg, e_st, num_segments=n)
    mip8 = jax.ops.segment_sum(pmsg, e_st, num_segments=n)

    pos8 = jnp.pad(pos, ((0, 0), (0, 5)))
    hw1a = fh_w1[:_H]
    hw1b = fh_w1[_H:]
    pw1 = jnp.pad(fhpos_w1, ((0, 2), (0, 5)))
    pb1 = jnp.pad(fhpos_b1.reshape(1, 3), ((0, 0), (0, 5)))
    pw2 = jnp.pad(fhpos_w2, ((0, 5), (0, 5)))
    pb2n = jnp.pad(fhpos_b2.reshape(1, 3), ((0, 0), (0, 5)))
    x_new, pos_new8 = _node_mlp(x, m_i, pos8, mip8, hw1a, hw1b,
                                fh_b1.reshape(1, _H), fh_w2,
                                fh_b2.reshape(1, _H), pw1, pb1, pw2, pb2n)
    return (x_new, edge_index, pos_new8[:, 0:3])


# speedup vs baseline: 2.7637x; 2.7637x over previous
import functools

import jax
import jax.numpy as jnp
from jax import lax
from jax.experimental import pallas as pl
from jax.experimental.pallas import tpu as pltpu
from jax.experimental.pallas import tpu_sc as plsc

F32 = jnp.float32
I32 = jnp.int32
_H = 256
_NW = 32
_CHUNK = 128


def _silu(v):
    return v * jax.nn.sigmoid(v)


def _sc_mesh():
    return plsc.VectorSubcoreMesh(core_axis_name="c", subcore_axis_name="s")


_SC_PARAMS = pltpu.CompilerParams(needs_layout_passes=False)


def _premul_body(x_ref, w1a_ref, w1b_ref, b1_ref, wp_ref, bp_ref,
                 xa_ref, xb_ref, pp_ref):
    x = x_ref[...]
    xa_ref[...] = jnp.dot(x, w1a_ref[...], preferred_element_type=F32)
    xb_ref[...] = jnp.dot(x, w1b_ref[...], preferred_element_type=F32) + b1_ref[...]
    pp_ref[...] = jnp.dot(x, wp_ref[...], preferred_element_type=F32) + bp_ref[...]


def _premul(x, w1a, w1b, b1, wp, bp, nb=1000):
    n = x.shape[0]
    grid = (n // nb,)
    full = lambda r, c: pl.BlockSpec((r, c), lambda i: (0, 0))
    blk = lambda c: pl.BlockSpec((nb, c), lambda i: (i, 0))
    return pl.pallas_call(
        _premul_body,
        grid=grid,
        in_specs=[blk(_H), full(_H, _H), full(_H, _H), full(1, _H),
                  full(_H, 8), full(1, 8)],
        out_specs=[blk(_H), blk(_H), blk(8)],
        out_shape=[jax.ShapeDtypeStruct((n, _H), F32),
                   jax.ShapeDtypeStruct((n, _H), F32),
                   jax.ShapeDtypeStruct((n, 8), F32)],
    )(x, w1a, w1b, b1, wp, bp)


def _h1pre_body(nchunks, xa_hbm, xb_hbm, st_hbm, en_hbm, out_hbm,
                ia, ib, ba, bb, sem):
    c = lax.axis_index("c")
    s = lax.axis_index("s")
    wid = s * 2 + c
    per = nchunks // _NW
    extra = nchunks - per * _NW
    nch = jnp.where(wid < extra, per + 1, per)
    cbase = jnp.where(wid < extra, (per + 1) * wid,
                      extra * (per + 1) + per * (wid - extra))

    def chunk(ci, _):
        off = (cbase + ci) * _CHUNK
        pltpu.sync_copy(st_hbm.at[pl.ds(off, _CHUNK)], ia)
        pltpu.sync_copy(en_hbm.at[pl.ds(off, _CHUNK)], ib)
        pltpu.async_copy(xa_hbm.at[ia], ba, sem).wait()
        pltpu.async_copy(xb_hbm.at[ib], bb, sem).wait()

        def row(r, _):
            for k in range(_H // 16):
                sl = pl.ds(k * 16, 16)
                plsc.addupdate(ba.at[r, sl], bb[r, sl])
            return 0

        lax.fori_loop(0, _CHUNK, row, 0)
        pltpu.sync_copy(ba, out_hbm.at[pl.ds(off, _CHUNK)])
        return 0

    lax.fori_loop(0, nch, chunk, 0)


def _sc_gather_h1pre(xa, xb, st, en):
    e = st.shape[0]
    nchunks = e // _CHUNK
    body = functools.partial(_h1pre_body, nchunks)
    f = pl.kernel(
        body,
        out_type=jax.ShapeDtypeStruct((e, _H), F32),
        mesh=_sc_mesh(),
        compiler_params=_SC_PARAMS,
        scratch_types=[
            pltpu.VMEM((_CHUNK,), I32),
            pltpu.VMEM((_CHUNK,), I32),
            pltpu.VMEM((_CHUNK, _H), F32),
            pltpu.VMEM((_CHUNK, _H), F32),
            pltpu.SemaphoreType.DMA,
        ],
    )
    return f(xa, xb, st, en)


_GC = 1000


def _geom_body(n3, xtra_hbm, st_hbm, en_hbm, gout_hbm,
               tp, ta, tb, ivs, ive, gbuf):
    c = lax.axis_index("c")
    s = lax.axis_index("s")
    wid = s * 2 + c
    pltpu.sync_copy(xtra_hbm.at[pl.ds(0, n3)], tp)
    pltpu.sync_copy(xtra_hbm.at[pl.ds(n3, n3)], ta)
    pltpu.sync_copy(xtra_hbm.at[pl.ds(2 * n3, n3)], tb)
    iota = lax.iota(I32, 16)

    def chunk(ci, _):
        off = (wid * 5 + ci) * _GC
        pltpu.sync_copy(st_hbm.at[pl.ds(off, _GC)], ivs)
        pltpu.sync_copy(en_hbm.at[pl.ds(off, _GC)], ive)

        def group(g, _):
            o = jnp.minimum(g * 16, _GC - 16)
            isv = ivs[pl.ds(o, 16)] * 3
            iev = ive[pl.ds(o, 16)] * 3
            dx = plsc.load_gather(tp, [isv]) - plsc.load_gather(tp, [iev])
            dy = plsc.load_gather(tp, [isv + 1]) - plsc.load_gather(tp, [iev + 1])
            dz = plsc.load_gather(tp, [isv + 2]) - plsc.load_gather(tp, [iev + 2])
            d2 = dx * dx + dy * dy + dz * dz
            ax = plsc.load_gather(ta, [isv]) + plsc.load_gather(tb, [iev])
            ay = plsc.load_gather(ta, [isv + 1]) + plsc.load_gather(tb, [iev + 1])
            az = plsc.load_gather(ta, [isv + 2]) + plsc.load_gather(tb, [iev + 2])
            gb = (o + iota) * 8
            plsc.store_scatter(gbuf, [gb], d2)
            plsc.store_scatter(gbuf, [gb + 1], ax)
            plsc.store_scatter(gbuf, [gb + 2], ay)
            plsc.store_scatter(gbuf, [gb + 3], az)
            return 0

        lax.fori_loop(0, (_GC + 15) // 16, group, 0)
        pltpu.sync_copy(gbuf, gout_hbm.at[pl.ds(off * 8, _GC * 8)])
        return 0

    lax.fori_loop(0, 5, chunk, 0)


def _sc_geom(xtra, st, en, n):
    e = st.shape[0]
    n3 = 3 * n
    body = functools.partial(_geom_body, n3)
    f = pl.kernel(
        body,
        out_type=jax.ShapeDtypeStruct((e * 8,), F32),
        mesh=_sc_mesh(),
        compiler_params=_SC_PARAMS,
        scratch_types=[
            pltpu.VMEM((n3,), F32),
            pltpu.VMEM((n3,), F32),
            pltpu.VMEM((n3,), F32),
            pltpu.VMEM((_GC,), I32),
            pltpu.VMEM((_GC,), I32),
            pltpu.VMEM((_GC * 8,), F32),
        ],
    )
    return f(xtra, st, en)


def _edge_body(h1pre_ref, geom_ref, w1c_ref, w2_ref, b2_ref, finf_ref,
               finfb_ref, wpc_ref, wp2_ref, bp2_ref, inv_n_ref,
               msgl_ref, msgr_ref, pmsg_ref):
    d2 = geom_ref[:, 0:1]
    dist = jnp.sqrt(d2)
    h1 = _silu(h1pre_ref[...] + dist * w1c_ref[...])
    m = _silu(jnp.dot(h1, w2_ref[...], preferred_element_type=F32) + b2_ref[...])
    logit = jnp.sum(m * finf_ref[...], axis=1, keepdims=True) + finfb_ref[...]
    msg = jax.nn.sigmoid(logit) * m
    msgl_ref[...] = msg[:, 0:_H // 2]
    msgr_ref[...] = msg[:, _H // 2:]
    ps = geom_ref[:, 1:4]
    p1 = _silu(ps + dist * wpc_ref[0:1, 0:3])
    mpos = (p1[:, 0:1] * wp2_ref[0:1, 0:3]
            + p1[:, 1:2] * wp2_ref[1:2, 0:3]
            + p1[:, 2:3] * wp2_ref[2:3, 0:3]) + bp2_ref[0:1, 0:3]
    mpos = _silu(mpos)
    sc = dist * inv_n_ref[0, 0]
    pmsg_ref[...] = jnp.concatenate(
        [sc * mpos, jnp.zeros((mpos.shape[0], 5), F32)], axis=1)


def _edge_mlp(h1pre, geom, w1c, w2, b2, finf_row, finf_b, wpc, wp2, bp2,
              inv_n, eb=1000):
    e = h1pre.shape[0]
    grid = (e // eb,)
    full = lambda r, c: pl.BlockSpec((r, c), lambda i: (0, 0))
    blk = lambda c: pl.BlockSpec((eb, c), lambda i: (i, 0))
    return pl.pallas_call(
        _edge_body,
        grid=grid,
        in_specs=[blk(_H), blk(8), full(1, _H), full(_H, _H), full(1, _H),
                  full(1, _H), full(1, 1), full(1, 8), full(8, 8),
                  full(1, 8), full(1, 1)],
        out_specs=[blk(_H // 2), blk(_H // 2), blk(8)],
        out_shape=[jax.ShapeDtypeStruct((e, _H // 2), F32),
                   jax.ShapeDtypeStruct((e, _H // 2), F32),
                   jax.ShapeDtypeStruct((e, 8), F32)],
    )(h1pre, geom, w1c, w2, b2, finf_row, finf_b, wpc, wp2, bp2, inv_n)


def _scatter_body(e, n, msgl_hbm, msgr_hbm, pmsg_hbm, st_hbm, z_hbm, zp_hbm,
                  mil_hbm, mir_hbm, mip_hbm, ebuf, libuf, mbuf, pbuf,
                  acc, accp):
    c = lax.axis_index("c")
    s = lax.axis_index("s")
    half = n // 2
    rows = acc.shape[0]
    zper = rows // 16
    hw = _H // 2

    nchunks = e // _CHUNK
    per = nchunks // 16
    extra = nchunks - per * 16
    nch = jnp.where(s < extra, per + 1, per)
    cbase = jnp.where(s < extra, (per + 1) * s,
                      extra * (per + 1) + per * (s - extra))
    cper = (half // 16) // 8 * 8
    rem = half - 16 * cper

    def one_pass(p, _):
        lo = p * half
        pltpu.sync_copy(z_hbm.at[pl.ds(s * zper, zper)],
                        acc.at[pl.ds(s * zper, zper)])
        pltpu.sync_copy(zp_hbm.at[pl.ds(s * zper, zper)],
                        accp.at[pl.ds(s * zper, zper)])
        plsc.subcore_barrier()

        def chunk(ci, _):
            off = (cbase + ci) * _CHUNK
            pltpu.sync_copy(st_hbm.at[pl.ds(off, _CHUNK)], ebuf)

            @pl.when(c == 0)
            def _():
                pltpu.sync_copy(msgl_hbm.at[pl.ds(off, _CHUNK)], mbuf)
                pltpu.sync_copy(pmsg_hbm.at[pl.ds(off, _CHUNK)], pbuf)

            @pl.when(c == 1)
            def _():
                pltpu.sync_copy(msgr_hbm.at[pl.ds(off, _CHUNK)], mbuf)

            for g in range(_CHUNK // 16):
                sl = pl.ds(g * 16, 16)
                li = ebuf[sl] - lo
                ok = (li >= 0) & (li < half)
                libuf[sl] = jnp.where(ok, li, half)
            pltpu.sync_copy(mbuf, acc.at[libuf], add=True)

            @pl.when(c == 0)
            def _():
                pltpu.sync_copy(pbuf, accp.at[libuf], add=True)

            return 0

        lax.fori_loop(0, nch, chunk, 0)
        plsc.subcore_barrier()

        @pl.when(c == 0)
        def _():
            pltpu.sync_copy(acc.at[pl.ds(s * cper, cper)],
                            mil_hbm.at[pl.ds(lo + s * cper, cper)])
            pltpu.sync_copy(accp.at[pl.ds(s * cper, cper)],
                            mip_hbm.at[pl.ds(lo + s * cper, cper)])

            @pl.when((s == 0) & (rem > 0))
            def _():
                pltpu.sync_copy(acc.at[pl.ds(16 * cper, rem)],
                                mil_hbm.at[pl.ds(lo + 16 * cper, rem)])
                pltpu.sync_copy(accp.at[pl.ds(16 * cper, rem)],
                                mip_hbm.at[pl.ds(lo + 16 * cper, rem)])

        @pl.when(c == 1)
        def _():
            pltpu.sync_copy(acc.at[pl.ds(s * cper, cper)],
                            mir_hbm.at[pl.ds(lo + s * cper, cper)])

            @pl.when((s == 0) & (rem > 0))
            def _():
                pltpu.sync_copy(acc.at[pl.ds(16 * cper, rem)],
                                mir_hbm.at[pl.ds(lo + 16 * cper, rem)])

        plsc.subcore_barrier()
        return 0

    lax.fori_loop(0, 2, one_pass, 0)


def _sc_scatter(msgl, msgr, pmsg, st, n):
    e = st.shape[0]
    half = n // 2
    rows = ((half + 1 + 255) // 256) * 256
    hw = _H // 2
    z = jnp.zeros((rows, hw), F32)
    zp = jnp.zeros((rows, 8), F32)
    body = functools.partial(_scatter_body, e, n)
    f = pl.kernel(
        body,
        out_type=[jax.ShapeDtypeStruct((n, hw), F32),
                  jax.ShapeDtypeStruct((n, hw), F32),
                  jax.ShapeDtypeStruct((n, 8), F32)],
        mesh=_sc_mesh(),
        compiler_params=_SC_PARAMS,
        scratch_types=[
            pltpu.VMEM((_CHUNK,), I32),
            pltpu.VMEM((_CHUNK,), I32),
            pltpu.VMEM((_CHUNK, hw), F32),
            pltpu.VMEM((_CHUNK, 8), F32),
            pltpu.VMEM_SHARED((rows, hw), F32),
            pltpu.VMEM_SHARED((rows, 8), F32),
        ],
    )
    return f(msgl, msgr, pmsg, st, z, zp)


def _node_body(x_ref, mil_ref, mir_ref, pos_ref, mip_ref, w1a_ref,
               w1bl_ref, w1br_ref, b1_ref,
               w2_ref, b2_ref, pw1_ref, pb1_ref, pw2_ref, pb2_ref,
               xn_ref, pn_ref):
    x = x_ref[...]
    h = (jnp.dot(x, w1a_ref[...], preferred_element_type=F32)
         + jnp.dot(mil_ref[...], w1bl_ref[...], preferred_element_type=F32)
         + jnp.dot(mir_ref[...], w1br_ref[...], preferred_element_type=F32)
         + b1_ref[...])
    h = _silu(h)
    xn_ref[...] = x + jnp.dot(h, w2_ref[...], preferred_element_type=F32) + b2_ref[...]
    pos3 = pos_ref[:, 0:3]
    mip3 = mip_ref[:, 0:3]
    hp = pb1_ref[0:1, 0:3]
    for k in range(3):
        hp = hp + pos3[:, k:k + 1] * pw1_ref[k:k + 1, 0:3]
        hp = hp + mip3[:, k:k + 1] * pw1_ref[k + 3:k + 4, 0:3]
    p1 = _silu(hp)
    pd = (p1[:, 0:1] * pw2_ref[0:1, 0:3]
          + p1[:, 1:2] * pw2_ref[1:2, 0:3]
          + p1[:, 2:3] * pw2_ref[2:3, 0:3]) + pb2_ref[0:1, 0:3]
    pn_ref[...] = jnp.concatenate(
        [pos3 + pd, jnp.zeros((pd.shape[0], 5), F32)], axis=1)


def _node_mlp(x, mil, mir, pos8, mip8, w1a, w1bl, w1br, b1, w2, b2,
              pw1, pb1, pw2, pb2, nb=1000):
    n = x.shape[0]
    grid = (n // nb,)
    full = lambda r, c: pl.BlockSpec((r, c), lambda i: (0, 0))
    blk = lambda c: pl.BlockSpec((nb, c), lambda i: (i, 0))
    return pl.pallas_call(
        _node_body,
        grid=grid,
        in_specs=[blk(_H), blk(_H // 2), blk(_H // 2), blk(8), blk(8),
                  full(_H, _H), full(_H // 2, _H), full(_H // 2, _H),
                  full(1, _H), full(_H, _H), full(1, _H),
                  full(8, 8), full(1, 8), full(8, 8), full(1, 8)],
        out_specs=[blk(_H), blk(8)],
        out_shape=[jax.ShapeDtypeStruct((n, _H), F32),
                   jax.ShapeDtypeStruct((n, 8), F32)],
    )(x, mil, mir, pos8, mip8, w1a, w1bl, w1br, b1, w2, b2, pw1, pb1,
      pw2, pb2)


def kernel(x, edge_index, pos,
           fe_w1, fe_b1, fe_w2, fe_b2,
           finf_w, finf_b,
           fh_w1, fh_b1, fh_w2, fh_b2,
           fpos_w1, fpos_b1, fpos_w2, fpos_b2,
           fhpos_w1, fhpos_b1, fhpos_w2, fhpos_b2):
    n = x.shape[0]
    e = edge_index.shape[0]
    e_st = edge_index[:, 0].astype(I32)
    e_end = edge_index[:, 1].astype(I32)

    w1a = fe_w1[:_H]
    w1b = fe_w1[_H:2 * _H]
    w1c = fe_w1[2 * _H:2 * _H + 1]
    b1 = fe_b1.reshape(1, _H)
    wpa = fpos_w1[:_H]
    wpb = fpos_w1[_H:2 * _H]
    wpc = jnp.pad(fpos_w1[2 * _H:2 * _H + 1], ((0, 0), (0, 5)))
    wp = jnp.pad(jnp.concatenate([wpa, wpb], axis=1), ((0, 0), (0, 2)))
    bp = jnp.pad(jnp.concatenate(
        [jnp.zeros((3,), F32), fpos_b1]).reshape(1, 6), ((0, 0), (0, 2)))
    wp2 = jnp.pad(fpos_w2, ((0, 5), (0, 5)))
    bp2 = jnp.pad(fpos_b2.reshape(1, 3), ((0, 0), (0, 5)))
    finf_row = finf_w.reshape(1, _H)
    finfb = finf_b.reshape(1, 1)
    inv_n = jnp.full((1, 1), 1.0 / n, F32)

    xa, xb, pp = _premul(x, w1a, w1b, b1, wp, bp)

    h1pre = _sc_gather_h1pre(xa, xb, e_st, e_end)
    xtra = jnp.concatenate(
        [pos.reshape(-1), pp[:, 0:3].reshape(-1), pp[:, 3:6].reshape(-1)])
    geom = _sc_geom(xtra, e_st, e_end, n).reshape(e, 8)

    msgl, msgr, pmsg = _edge_mlp(h1pre, geom, w1c, fe_w2,
                                 fe_b2.reshape(1, _H),
                                 finf_row, finfb, wpc, wp2, bp2, inv_n)

    mil, mir, mip8 = _sc_scatter(msgl, msgr, pmsg, e_st, n)

    pos8 = jnp.pad(pos, ((0, 0), (0, 5)))
    pw1 = jnp.pad(fhpos_w1, ((0, 2), (0, 5)))
    pb1 = jnp.pad(fhpos_b1.reshape(1, 3), ((0, 0), (0, 5)))
    pw2 = jnp.pad(fhpos_w2, ((0, 5), (0, 5)))
    pb2n = jnp.pad(fhpos_b2.reshape(1, 3), ((0, 0), (0, 5)))
    x_new, pos_new8 = _node_mlp(x, mil, mir, pos8, mip8, fh_w1[:_H],
                                fh_w1[_H:_H + _H // 2],
                                fh_w1[_H + _H // 2:],
                                fh_b1.reshape(1, _H), fh_w2,
                                fh_b2.reshape(1, _H), pw1, pb1, pw2, pb2n)
    return (x_new, edge_index, pos_new8[:, 0:3])


# speedup vs baseline: 2.7683x; 1.0017x over previous
import functools

import jax
import jax.numpy as jnp
from jax import lax
from jax.experimental import pallas as pl
from jax.experimental.pallas import tpu as pltpu
from jax.experimental.pallas import tpu_sc as plsc

F32 = jnp.float32
I32 = jnp.int32
_H = 256
_NW = 32
_CHUNK = 128


def _silu(v):
    return v * jax.nn.sigmoid(v)


def _sc_mesh():
    return plsc.VectorSubcoreMesh(core_axis_name="c", subcore_axis_name="s")


_SC_PARAMS = pltpu.CompilerParams(needs_layout_passes=False)


def _premul_body(x_ref, w1a_ref, w1b_ref, b1_ref, wp_ref, bp_ref,
                 xa_ref, xb_ref, pp_ref):
    x = x_ref[...]
    xa_ref[...] = jnp.dot(x, w1a_ref[...], preferred_element_type=F32)
    xb_ref[...] = jnp.dot(x, w1b_ref[...], preferred_element_type=F32) + b1_ref[...]
    pp_ref[...] = jnp.dot(x, wp_ref[...], preferred_element_type=F32) + bp_ref[...]


def _premul(x, w1a, w1b, b1, wp, bp, nb=1000):
    n = x.shape[0]
    grid = (n // nb,)
    full = lambda r, c: pl.BlockSpec((r, c), lambda i: (0, 0))
    blk = lambda c: pl.BlockSpec((nb, c), lambda i: (i, 0))
    return pl.pallas_call(
        _premul_body,
        grid=grid,
        in_specs=[blk(_H), full(_H, _H), full(_H, _H), full(1, _H),
                  full(_H, 8), full(1, 8)],
        out_specs=[blk(_H), blk(_H), blk(8)],
        out_shape=[jax.ShapeDtypeStruct((n, _H), F32),
                   jax.ShapeDtypeStruct((n, _H), F32),
                   jax.ShapeDtypeStruct((n, 8), F32)],
    )(x, w1a, w1b, b1, wp, bp)


def _h1pre_body(nchunks, xa_hbm, xb_hbm, st_hbm, en_hbm, out_hbm,
                ia, ib, ba, bb, sem):
    c = lax.axis_index("c")
    s = lax.axis_index("s")
    wid = s * 2 + c
    per = nchunks // _NW
    extra = nchunks - per * _NW
    nch = jnp.where(wid < extra, per + 1, per)
    cbase = jnp.where(wid < extra, (per + 1) * wid,
                      extra * (per + 1) + per * (wid - extra))

    def chunk(ci, _):
        off = (cbase + ci) * _CHUNK
        pltpu.sync_copy(st_hbm.at[pl.ds(off, _CHUNK)], ia)
        pltpu.sync_copy(en_hbm.at[pl.ds(off, _CHUNK)], ib)
        pltpu.async_copy(xa_hbm.at[ia], ba, sem).wait()
        pltpu.async_copy(xb_hbm.at[ib], bb, sem).wait()

        def row(r, _):
            for k in range(_H // 16):
                sl = pl.ds(k * 16, 16)
                plsc.addupdate(ba.at[r, sl], bb[r, sl])
            return 0

        lax.fori_loop(0, _CHUNK, row, 0)
        pltpu.sync_copy(ba, out_hbm.at[pl.ds(off, _CHUNK)])
        return 0

    lax.fori_loop(0, nch, chunk, 0)


def _sc_gather_h1pre(xa, xb, st, en):
    e = st.shape[0]
    nchunks = e // _CHUNK
    body = functools.partial(_h1pre_body, nchunks)
    f = pl.kernel(
        body,
        out_type=jax.ShapeDtypeStruct((e, _H), F32),
        mesh=_sc_mesh(),
        compiler_params=_SC_PARAMS,
        scratch_types=[
            pltpu.VMEM((_CHUNK,), I32),
            pltpu.VMEM((_CHUNK,), I32),
            pltpu.VMEM((_CHUNK, _H), F32),
            pltpu.VMEM((_CHUNK, _H), F32),
            pltpu.SemaphoreType.DMA,
        ],
    )
    return f(xa, xb, st, en)


_GC = 1000


def _geom_body(n3, xtra_hbm, st_hbm, en_hbm, gout_hbm,
               tp, ta, tb, ivs, ive, gbuf):
    c = lax.axis_index("c")
    s = lax.axis_index("s")
    wid = s * 2 + c
    pltpu.sync_copy(xtra_hbm.at[pl.ds(0, n3)], tp)
    pltpu.sync_copy(xtra_hbm.at[pl.ds(n3, n3)], ta)
    pltpu.sync_copy(xtra_hbm.at[pl.ds(2 * n3, n3)], tb)
    iota = lax.iota(I32, 16)

    def chunk(ci, _):
        off = (wid * 5 + ci) * _GC
        pltpu.sync_copy(st_hbm.at[pl.ds(off, _GC)], ivs)
        pltpu.sync_copy(en_hbm.at[pl.ds(off, _GC)], ive)

        def group(g, _):
            o = jnp.minimum(g * 16, _GC - 16)
            isv = ivs[pl.ds(o, 16)] * 3
            iev = ive[pl.ds(o, 16)] * 3
            dx = plsc.load_gather(tp, [isv]) - plsc.load_gather(tp, [iev])
            dy = plsc.load_gather(tp, [isv + 1]) - plsc.load_gather(tp, [iev + 1])
            dz = plsc.load_gather(tp, [isv + 2]) - plsc.load_gather(tp, [iev + 2])
            d2 = dx * dx + dy * dy + dz * dz
            ax = plsc.load_gather(ta, [isv]) + plsc.load_gather(tb, [iev])
            ay = plsc.load_gather(ta, [isv + 1]) + plsc.load_gather(tb, [iev + 1])
            az = plsc.load_gather(ta, [isv + 2]) + plsc.load_gather(tb, [iev + 2])
            gb = (o + iota) * 8
            plsc.store_scatter(gbuf, [gb], d2)
            plsc.store_scatter(gbuf, [gb + 1], ax)
            plsc.store_scatter(gbuf, [gb + 2], ay)
            plsc.store_scatter(gbuf, [gb + 3], az)
            return 0

        lax.fori_loop(0, (_GC + 15) // 16, group, 0)
        pltpu.sync_copy(gbuf, gout_hbm.at[pl.ds(off * 8, _GC * 8)])
        return 0

    lax.fori_loop(0, 5, chunk, 0)


def _sc_geom(xtra, st, en, n):
    e = st.shape[0]
    n3 = 3 * n
    body = functools.partial(_geom_body, n3)
    f = pl.kernel(
        body,
        out_type=jax.ShapeDtypeStruct((e * 8,), F32),
        mesh=_sc_mesh(),
        compiler_params=_SC_PARAMS,
        scratch_types=[
            pltpu.VMEM((n3,), F32),
            pltpu.VMEM((n3,), F32),
            pltpu.VMEM((n3,), F32),
            pltpu.VMEM((_GC,), I32),
            pltpu.VMEM((_GC,), I32),
            pltpu.VMEM((_GC * 8,), F32),
        ],
    )
    return f(xtra, st, en)


def _edge_body(h1pre_ref, geom_ref, w1c_ref, w2_ref, b2_ref, finf_ref,
               finfb_ref, wpc_ref, wp2_ref, bp2_ref, inv_n_ref,
               msgl_ref, msgr_ref, pmsg_ref):
    d2 = geom_ref[:, 0:1]
    dist = jnp.sqrt(d2)
    h1 = _silu(h1pre_ref[...] + dist * w1c_ref[...])
    m = _silu(jnp.dot(h1, w2_ref[...], preferred_element_type=F32) + b2_ref[...])
    logit = jnp.sum(m * finf_ref[...], axis=1, keepdims=True) + finfb_ref[...]
    msg = jax.nn.sigmoid(logit) * m
    msgl_ref[...] = msg[:, 0:_H // 2]
    msgr_ref[...] = msg[:, _H // 2:]
    ps = geom_ref[:, 1:4]
    p1 = _silu(ps + dist * wpc_ref[0:1, 0:3])
    mpos = (p1[:, 0:1] * wp2_ref[0:1, 0:3]
            + p1[:, 1:2] * wp2_ref[1:2, 0:3]
            + p1[:, 2:3] * wp2_ref[2:3, 0:3]) + bp2_ref[0:1, 0:3]
    mpos = _silu(mpos)
    sc = dist * inv_n_ref[0, 0]
    pmsg_ref[...] = jnp.concatenate(
        [sc * mpos, jnp.zeros((mpos.shape[0], 5), F32)], axis=1)


def _edge_mlp(h1pre, geom, w1c, w2, b2, finf_row, finf_b, wpc, wp2, bp2,
              inv_n, eb=1000):
    e = h1pre.shape[0]
    grid = (e // eb,)
    full = lambda r, c: pl.BlockSpec((r, c), lambda i: (0, 0))
    blk = lambda c: pl.BlockSpec((eb, c), lambda i: (i, 0))
    return pl.pallas_call(
        _edge_body,
        grid=grid,
        in_specs=[blk(_H), blk(8), full(1, _H), full(_H, _H), full(1, _H),
                  full(1, _H), full(1, 1), full(1, 8), full(8, 8),
                  full(1, 8), full(1, 1)],
        out_specs=[blk(_H // 2), blk(_H // 2), blk(8)],
        out_shape=[jax.ShapeDtypeStruct((e, _H // 2), F32),
                   jax.ShapeDtypeStruct((e, _H // 2), F32),
                   jax.ShapeDtypeStruct((e, 8), F32)],
    )(h1pre, geom, w1c, w2, b2, finf_row, finf_b, wpc, wp2, bp2, inv_n)


def _scatter_body(e, n, msgl_hbm, msgr_hbm, pmsg_hbm, st_hbm, z_hbm, zp_hbm,
                  mil_hbm, mir_hbm, mip_hbm, ebig,
                  libuf0, libuf1, mbuf0, mbuf1, pbuf0, pbuf1,
                  lsem0, lsem1, acc, accp):
    del ebig, mbuf1, pbuf1, lsem0, lsem1
    ebuf, libuf, mbuf, pbuf = libuf0, libuf1, mbuf0, pbuf0
    c = lax.axis_index("c")
    s = lax.axis_index("s")
    half = n // 2
    rows = acc.shape[0]
    zper = rows // 16
    hw = _H // 2

    nchunks = e // _CHUNK
    per = nchunks // 16
    extra = nchunks - per * 16
    nch = jnp.where(s < extra, per + 1, per)
    cbase = jnp.where(s < extra, (per + 1) * s,
                      extra * (per + 1) + per * (s - extra))
    cper = (half // 16) // 8 * 8
    rem = half - 16 * cper

    def one_pass(p, _):
        lo = p * half
        pm = c == 0
        pltpu.sync_copy(z_hbm.at[pl.ds(s * zper, zper)],
                        acc.at[pl.ds(s * zper, zper)])
        pltpu.sync_copy(zp_hbm.at[pl.ds(s * zper, zper)],
                        accp.at[pl.ds(s * zper, zper)])
        plsc.subcore_barrier()

        dump = half + lax.iota(I32, 16)

        def chunk(ci, _):
            off = (cbase + ci) * _CHUNK
            pltpu.sync_copy(st_hbm.at[pl.ds(off, _CHUNK)], ebuf)

            @pl.when(c == 0)
            def _():
                pltpu.sync_copy(msgl_hbm.at[pl.ds(off, _CHUNK)], mbuf)

            @pl.when(c == 1)
            def _():
                pltpu.sync_copy(msgr_hbm.at[pl.ds(off, _CHUNK)], mbuf)

            @pl.when(pm)
            def _():
                pltpu.sync_copy(pmsg_hbm.at[pl.ds(off, _CHUNK)], pbuf)

            for g in range(_CHUNK // 16):
                sl = pl.ds(g * 16, 16)
                li = ebuf[sl] - lo
                ok = (li >= 0) & (li < half)
                libuf[sl] = jnp.where(ok, li, dump)
            pltpu.sync_copy(mbuf, acc.at[libuf], add=True)

            @pl.when(pm)
            def _():
                pltpu.sync_copy(pbuf, accp.at[libuf], add=True)

            return 0

        lax.fori_loop(0, nch, chunk, 0)
        plsc.subcore_barrier()

        @pl.when(c == 0)
        def _():
            pltpu.sync_copy(acc.at[pl.ds(s * cper, cper)],
                            mil_hbm.at[pl.ds(lo + s * cper, cper)])

            @pl.when((s == 0) & (rem > 0))
            def _():
                pltpu.sync_copy(acc.at[pl.ds(16 * cper, rem)],
                                mil_hbm.at[pl.ds(lo + 16 * cper, rem)])

        @pl.when(c == 1)
        def _():
            pltpu.sync_copy(acc.at[pl.ds(s * cper, cper)],
                            mir_hbm.at[pl.ds(lo + s * cper, cper)])

            @pl.when((s == 0) & (rem > 0))
            def _():
                pltpu.sync_copy(acc.at[pl.ds(16 * cper, rem)],
                                mir_hbm.at[pl.ds(lo + 16 * cper, rem)])

        @pl.when(pm)
        def _():
            pltpu.sync_copy(accp.at[pl.ds(s * cper, cper)],
                            mip_hbm.at[pl.ds(lo + s * cper, cper)])

            @pl.when((s == 0) & (rem > 0))
            def _():
                pltpu.sync_copy(accp.at[pl.ds(16 * cper, rem)],
                                mip_hbm.at[pl.ds(lo + 16 * cper, rem)])

        plsc.subcore_barrier()
        return 0

    lax.fori_loop(0, 2, one_pass, 0)


def _sc_scatter(msgl, msgr, pmsg, st, n):
    e = st.shape[0]
    half = n // 2
    rows = ((half + 1 + 255) // 256) * 256
    hw = _H // 2
    z = jnp.zeros((rows, hw), F32)
    zp = jnp.zeros((rows, 8), F32)
    body = functools.partial(_scatter_body, e, n)
    f = pl.kernel(
        body,
        out_type=[jax.ShapeDtypeStruct((n, hw), F32),
                  jax.ShapeDtypeStruct((n, hw), F32),
                  jax.ShapeDtypeStruct((n, 8), F32)],
        mesh=_sc_mesh(),
        compiler_params=_SC_PARAMS,
        scratch_types=[
            pltpu.VMEM((80 * _CHUNK,), I32),
            pltpu.VMEM((_CHUNK,), I32),
            pltpu.VMEM((_CHUNK,), I32),
            pltpu.VMEM((_CHUNK, hw), F32),
            pltpu.VMEM((_CHUNK, hw), F32),
            pltpu.VMEM((_CHUNK, 8), F32),
            pltpu.VMEM((_CHUNK, 8), F32),
            pltpu.SemaphoreType.DMA,
            pltpu.SemaphoreType.DMA,
            pltpu.VMEM_SHARED((rows, hw), F32),
            pltpu.VMEM_SHARED((rows, 8), F32),
        ],
    )
    return f(msgl, msgr, pmsg, st, z, zp)


def _node_body(x_ref, mil_ref, mir_ref, pos_ref, mip_ref, w1a_ref,
               w1bl_ref, w1br_ref, b1_ref,
               w2_ref, b2_ref, pw1_ref, pb1_ref, pw2_ref, pb2_ref,
               xn_ref, pn_ref):
    x = x_ref[...]
    h = (jnp.dot(x, w1a_ref[...], preferred_element_type=F32)
         + jnp.dot(mil_ref[...], w1bl_ref[...], preferred_element_type=F32)
         + jnp.dot(mir_ref[...], w1br_ref[...], preferred_element_type=F32)
         + b1_ref[...])
    h = _silu(h)
    xn_ref[...] = x + jnp.dot(h, w2_ref[...], preferred_element_type=F32) + b2_ref[...]
    pos3 = pos_ref[:, 0:3]
    mip3 = mip_ref[:, 0:3]
    hp = pb1_ref[0:1, 0:3]
    for k in range(3):
        hp = hp + pos3[:, k:k + 1] * pw1_ref[k:k + 1, 0:3]
        hp = hp + mip3[:, k:k + 1] * pw1_ref[k + 3:k + 4, 0:3]
    p1 = _silu(hp)
    pd = (p1[:, 0:1] * pw2_ref[0:1, 0:3]
          + p1[:, 1:2] * pw2_ref[1:2, 0:3]
          + p1[:, 2:3] * pw2_ref[2:3, 0:3]) + pb2_ref[0:1, 0:3]
    pn_ref[...] = jnp.concatenate(
        [pos3 + pd, jnp.zeros((pd.shape[0], 5), F32)], axis=1)


def _node_mlp(x, mil, mir, pos8, mip8, w1a, w1bl, w1br, b1, w2, b2,
              pw1, pb1, pw2, pb2, nb=1000):
    n = x.shape[0]
    grid = (n // nb,)
    full = lambda r, c: pl.BlockSpec((r, c), lambda i: (0, 0))
    blk = lambda c: pl.BlockSpec((nb, c), lambda i: (i, 0))
    return pl.pallas_call(
        _node_body,
        grid=grid,
        in_specs=[blk(_H), blk(_H // 2), blk(_H // 2), blk(8), blk(8),
                  full(_H, _H), full(_H // 2, _H), full(_H // 2, _H),
                  full(1, _H), full(_H, _H), full(1, _H),
                  full(8, 8), full(1, 8), full(8, 8), full(1, 8)],
        out_specs=[blk(_H), blk(8)],
        out_shape=[jax.ShapeDtypeStruct((n, _H), F32),
                   jax.ShapeDtypeStruct((n, 8), F32)],
    )(x, mil, mir, pos8, mip8, w1a, w1bl, w1br, b1, w2, b2, pw1, pb1,
      pw2, pb2)


def kernel(x, edge_index, pos,
           fe_w1, fe_b1, fe_w2, fe_b2,
           finf_w, finf_b,
           fh_w1, fh_b1, fh_w2, fh_b2,
           fpos_w1, fpos_b1, fpos_w2, fpos_b2,
           fhpos_w1, fhpos_b1, fhpos_w2, fhpos_b2):
    n = x.shape[0]
    e = edge_index.shape[0]
    e_st = edge_index[:, 0].astype(I32)
    e_end = edge_index[:, 1].astype(I32)

    w1a = fe_w1[:_H]
    w1b = fe_w1[_H:2 * _H]
    w1c = fe_w1[2 * _H:2 * _H + 1]
    b1 = fe_b1.reshape(1, _H)
    wpa = fpos_w1[:_H]
    wpb = fpos_w1[_H:2 * _H]
    wpc = jnp.pad(fpos_w1[2 * _H:2 * _H + 1], ((0, 0), (0, 5)))
    wp = jnp.pad(jnp.concatenate([wpa, wpb], axis=1), ((0, 0), (0, 2)))
    bp = jnp.pad(jnp.concatenate(
        [jnp.zeros((3,), F32), fpos_b1]).reshape(1, 6), ((0, 0), (0, 2)))
    wp2 = jnp.pad(fpos_w2, ((0, 5), (0, 5)))
    bp2 = jnp.pad(fpos_b2.reshape(1, 3), ((0, 0), (0, 5)))
    finf_row = finf_w.reshape(1, _H)
    finfb = finf_b.reshape(1, 1)
    inv_n = jnp.full((1, 1), 1.0 / n, F32)

    xa, xb, pp = _premul(x, w1a, w1b, b1, wp, bp)

    h1pre = _sc_gather_h1pre(xa, xb, e_st, e_end)
    xtra = jnp.concatenate(
        [pos.reshape(-1), pp[:, 0:3].reshape(-1), pp[:, 3:6].reshape(-1)])
    geom = _sc_geom(xtra, e_st, e_end, n).reshape(e, 8)

    msgl, msgr, pmsg = _edge_mlp(h1pre, geom, w1c, fe_w2,
                                 fe_b2.reshape(1, _H),
                                 finf_row, finfb, wpc, wp2, bp2, inv_n)

    mil, mir, mip8 = _sc_scatter(msgl, msgr, pmsg, e_st, n)

    pos8 = jnp.pad(pos, ((0, 0), (0, 5)))
    pw1 = jnp.pad(fhpos_w1, ((0, 2), (0, 5)))
    pb1 = jnp.pad(fhpos_b1.reshape(1, 3), ((0, 0), (0, 5)))
    pw2 = jnp.pad(fhpos_w2, ((0, 5), (0, 5)))
    pb2n = jnp.pad(fhpos_b2.reshape(1, 3), ((0, 0), (0, 5)))
    x_new, pos_new8 = _node_mlp(x, mil, mir, pos8, mip8, fh_w1[:_H],
                                fh_w1[_H:_H + _H // 2],
                                fh_w1[_H + _H // 2:],
                                fh_b1.reshape(1, _H), fh_w2,
                                fh_b2.reshape(1, _H), pw1, pb1, pw2, pb2n)
    return (x_new, edge_index, pos_new8[:, 0:3])


# speedup vs baseline: 2.8193x; 1.0184x over previous
import functools

import jax
import jax.numpy as jnp
from jax import lax
from jax.experimental import pallas as pl
from jax.experimental.pallas import tpu as pltpu
from jax.experimental.pallas import tpu_sc as plsc

F32 = jnp.float32
I32 = jnp.int32
_H = 256
_NW = 32
_CHUNK = 128


def _silu(v):
    return v * jax.nn.sigmoid(v)


def _sc_mesh():
    return plsc.VectorSubcoreMesh(core_axis_name="c", subcore_axis_name="s")


_SC_PARAMS = pltpu.CompilerParams(needs_layout_passes=False)


def _premul_body(x_ref, w1a_ref, w1b_ref, b1_ref, wp_ref, bp_ref,
                 xa_ref, xb_ref, pp_ref):
    x = x_ref[...]
    xa_ref[...] = jnp.dot(x, w1a_ref[...], preferred_element_type=F32)
    xb_ref[...] = jnp.dot(x, w1b_ref[...], preferred_element_type=F32) + b1_ref[...]
    pp_ref[...] = jnp.dot(x, wp_ref[...], preferred_element_type=F32) + bp_ref[...]


def _premul(x, w1a, w1b, b1, wp, bp, nb=1000):
    n = x.shape[0]
    grid = (n // nb,)
    full = lambda r, c: pl.BlockSpec((r, c), lambda i: (0, 0))
    blk = lambda c: pl.BlockSpec((nb, c), lambda i: (i, 0))
    return pl.pallas_call(
        _premul_body,
        grid=grid,
        in_specs=[blk(_H), full(_H, _H), full(_H, _H), full(1, _H),
                  full(_H, 8), full(1, 8)],
        out_specs=[blk(_H), blk(_H), blk(8)],
        out_shape=[jax.ShapeDtypeStruct((n, _H), F32),
                   jax.ShapeDtypeStruct((n, _H), F32),
                   jax.ShapeDtypeStruct((n, 8), F32)],
    )(x, w1a, w1b, b1, wp, bp)


def _h1pre_body(nchunks, xa_hbm, xb_hbm, st_hbm, en_hbm, out_hbm,
                ia, ib, ba, bb, sem):
    c = lax.axis_index("c")
    s = lax.axis_index("s")
    wid = s * 2 + c
    per = nchunks // _NW
    extra = nchunks - per * _NW
    nch = jnp.where(wid < extra, per + 1, per)
    cbase = jnp.where(wid < extra, (per + 1) * wid,
                      extra * (per + 1) + per * (wid - extra))

    def chunk(ci, _):
        off = (cbase + ci) * _CHUNK
        pltpu.sync_copy(st_hbm.at[pl.ds(off, _CHUNK)], ia)
        pltpu.sync_copy(en_hbm.at[pl.ds(off, _CHUNK)], ib)
        da = pltpu.async_copy(xa_hbm.at[ia], ba, sem)
        db = pltpu.async_copy(xb_hbm.at[ib], bb, sem)
        da.wait()
        db.wait()

        def row(r, _):
            for k in range(_H // 16):
                sl = pl.ds(k * 16, 16)
                plsc.addupdate(ba.at[r, sl], bb[r, sl])
            return 0

        lax.fori_loop(0, _CHUNK, row, 0)
        pltpu.sync_copy(ba, out_hbm.at[pl.ds(off, _CHUNK)])
        return 0

    lax.fori_loop(0, nch, chunk, 0)


def _sc_gather_h1pre(xa, xb, st, en):
    e = st.shape[0]
    nchunks = e // _CHUNK
    body = functools.partial(_h1pre_body, nchunks)
    f = pl.kernel(
        body,
        out_type=jax.ShapeDtypeStruct((e, _H), F32),
        mesh=_sc_mesh(),
        compiler_params=_SC_PARAMS,
        scratch_types=[
            pltpu.VMEM((_CHUNK,), I32),
            pltpu.VMEM((_CHUNK,), I32),
            pltpu.VMEM((_CHUNK, _H), F32),
            pltpu.VMEM((_CHUNK, _H), F32),
            pltpu.SemaphoreType.DMA,
        ],
    )
    return f(xa, xb, st, en)


_GC = 1000


def _geom_body(n3, xtra_hbm, st_hbm, en_hbm, gout_hbm,
               tp, ta, tb, ivs, ive, gbuf):
    c = lax.axis_index("c")
    s = lax.axis_index("s")
    wid = s * 2 + c
    pltpu.sync_copy(xtra_hbm.at[pl.ds(0, n3)], tp)
    pltpu.sync_copy(xtra_hbm.at[pl.ds(n3, n3)], ta)
    pltpu.sync_copy(xtra_hbm.at[pl.ds(2 * n3, n3)], tb)
    iota = lax.iota(I32, 16)

    def chunk(ci, _):
        off = (wid * 5 + ci) * _GC
        pltpu.sync_copy(st_hbm.at[pl.ds(off, _GC)], ivs)
        pltpu.sync_copy(en_hbm.at[pl.ds(off, _GC)], ive)

        def group(g, _):
            o = jnp.minimum(g * 16, _GC - 16)
            isv = ivs[pl.ds(o, 16)] * 3
            iev = ive[pl.ds(o, 16)] * 3
            dx = plsc.load_gather(tp, [isv]) - plsc.load_gather(tp, [iev])
            dy = plsc.load_gather(tp, [isv + 1]) - plsc.load_gather(tp, [iev + 1])
            dz = plsc.load_gather(tp, [isv + 2]) - plsc.load_gather(tp, [iev + 2])
            d2 = dx * dx + dy * dy + dz * dz
            ax = plsc.load_gather(ta, [isv]) + plsc.load_gather(tb, [iev])
            ay = plsc.load_gather(ta, [isv + 1]) + plsc.load_gather(tb, [iev + 1])
            az = plsc.load_gather(ta, [isv + 2]) + plsc.load_gather(tb, [iev + 2])
            gb = (o + iota) * 8
            plsc.store_scatter(gbuf, [gb], d2)
            plsc.store_scatter(gbuf, [gb + 1], ax)
            plsc.store_scatter(gbuf, [gb + 2], ay)
            plsc.store_scatter(gbuf, [gb + 3], az)
            return 0

        lax.fori_loop(0, (_GC + 15) // 16, group, 0)
        pltpu.sync_copy(gbuf, gout_hbm.at[pl.ds(off * 8, _GC * 8)])
        return 0

    lax.fori_loop(0, 5, chunk, 0)


def _sc_geom(xtra, st, en, n):
    e = st.shape[0]
    n3 = 3 * n
    body = functools.partial(_geom_body, n3)
    f = pl.kernel(
        body,
        out_type=jax.ShapeDtypeStruct((e * 8,), F32),
        mesh=_sc_mesh(),
        compiler_params=_SC_PARAMS,
        scratch_types=[
            pltpu.VMEM((n3,), F32),
            pltpu.VMEM((n3,), F32),
            pltpu.VMEM((n3,), F32),
            pltpu.VMEM((_GC,), I32),
            pltpu.VMEM((_GC,), I32),
            pltpu.VMEM((_GC * 8,), F32),
        ],
    )
    return f(xtra, st, en)


def _edge_body(h1pre_ref, geom_ref, w1c_ref, w2_ref, b2_ref, finf_ref,
               finfb_ref, wpc_ref, wp2_ref, bp2_ref, inv_n_ref,
               msgl_ref, msgr_ref, pmsg_ref):
    d2 = geom_ref[:, 0:1]
    dist = jnp.sqrt(d2)
    h1 = _silu(h1pre_ref[...] + dist * w1c_ref[...])
    m = _silu(jnp.dot(h1, w2_ref[...], preferred_element_type=F32) + b2_ref[...])
    logit = jnp.sum(m * finf_ref[...], axis=1, keepdims=True) + finfb_ref[...]
    msg = jax.nn.sigmoid(logit) * m
    msgl_ref[...] = msg[:, 0:_H // 2]
    msgr_ref[...] = msg[:, _H // 2:]
    ps = geom_ref[:, 1:4]
    p1 = _silu(ps + dist * wpc_ref[0:1, 0:3])
    mpos = (p1[:, 0:1] * wp2_ref[0:1, 0:3]
            + p1[:, 1:2] * wp2_ref[1:2, 0:3]
            + p1[:, 2:3] * wp2_ref[2:3, 0:3]) + bp2_ref[0:1, 0:3]
    mpos = _silu(mpos)
    sc = dist * inv_n_ref[0, 0]
    pmsg_ref[...] = jnp.concatenate(
        [sc * mpos, jnp.zeros((mpos.shape[0], 5), F32)], axis=1)


def _edge_mlp(h1pre, geom, w1c, w2, b2, finf_row, finf_b, wpc, wp2, bp2,
              inv_n, eb=1000):
    e = h1pre.shape[0]
    grid = (e // eb,)
    full = lambda r, c: pl.BlockSpec((r, c), lambda i: (0, 0))
    blk = lambda c: pl.BlockSpec((eb, c), lambda i: (i, 0))
    return pl.pallas_call(
        _edge_body,
        grid=grid,
        in_specs=[blk(_H), blk(8), full(1, _H), full(_H, _H), full(1, _H),
                  full(1, _H), full(1, 1), full(1, 8), full(8, 8),
                  full(1, 8), full(1, 1)],
        out_specs=[blk(_H // 2), blk(_H // 2), blk(8)],
        out_shape=[jax.ShapeDtypeStruct((e, _H // 2), F32),
                   jax.ShapeDtypeStruct((e, _H // 2), F32),
                   jax.ShapeDtypeStruct((e, 8), F32)],
    )(h1pre, geom, w1c, w2, b2, finf_row, finf_b, wpc, wp2, bp2, inv_n)


def _scatter_body(e, n, msgl_hbm, msgr_hbm, pmsg_hbm, st_hbm, z_hbm, zp_hbm,
                  mil_hbm, mir_hbm, mip_hbm, ebig,
                  libuf0, libuf1, mbuf0, mbuf1, pbuf0, pbuf1,
                  lsem0, lsem1, acc, accp):
    del ebig, mbuf1, pbuf1, lsem0, lsem1
    ebuf, libuf, mbuf, pbuf = libuf0, libuf1, mbuf0, pbuf0
    c = lax.axis_index("c")
    s = lax.axis_index("s")
    half = n // 2
    rows = acc.shape[0]
    zper = rows // 16
    hw = _H // 2

    nchunks = e // _CHUNK
    per = nchunks // 16
    extra = nchunks - per * 16
    nch = jnp.where(s < extra, per + 1, per)
    cbase = jnp.where(s < extra, (per + 1) * s,
                      extra * (per + 1) + per * (s - extra))
    cper = (half // 16) // 8 * 8
    rem = half - 16 * cper

    def one_pass(p, _):
        lo = p * half
        pm = c == 0
        pltpu.sync_copy(z_hbm.at[pl.ds(s * zper, zper)],
                        acc.at[pl.ds(s * zper, zper)])
        pltpu.sync_copy(zp_hbm.at[pl.ds(s * zper, zper)],
                        accp.at[pl.ds(s * zper, zper)])
        plsc.subcore_barrier()

        dump = half + lax.iota(I32, 16)

        def chunk(ci, _):
            off = (cbase + ci) * _CHUNK
            pltpu.sync_copy(st_hbm.at[pl.ds(off, _CHUNK)], ebuf)

            @pl.when(c == 0)
            def _():
                pltpu.sync_copy(msgl_hbm.at[pl.ds(off, _CHUNK)], mbuf)

            @pl.when(c == 1)
            def _():
                pltpu.sync_copy(msgr_hbm.at[pl.ds(off, _CHUNK)], mbuf)

            @pl.when(pm)
            def _():
                pltpu.sync_copy(pmsg_hbm.at[pl.ds(off, _CHUNK)], pbuf)

            for g in range(_CHUNK // 16):
                sl = pl.ds(g * 16, 16)
                li = ebuf[sl] - lo
                ok = (li >= 0) & (li < half)
                libuf[sl] = jnp.where(ok, li, dump)
            pltpu.sync_copy(mbuf, acc.at[libuf], add=True)

            @pl.when(pm)
            def _():
                pltpu.sync_copy(pbuf, accp.at[libuf], add=True)

            return 0

        lax.fori_loop(0, nch, chunk, 0)
        plsc.subcore_barrier()

        @pl.when(c == 0)
        def _():
            pltpu.sync_copy(acc.at[pl.ds(s * cper, cper)],
                            mil_hbm.at[pl.ds(lo + s * cper, cper)])

            @pl.when((s == 0) & (rem > 0))
            def _():
                pltpu.sync_copy(acc.at[pl.ds(16 * cper, rem)],
                                mil_hbm.at[pl.ds(lo + 16 * cper, rem)])

        @pl.when(c == 1)
        def _():
            pltpu.sync_copy(acc.at[pl.ds(s * cper, cper)],
                            mir_hbm.at[pl.ds(lo + s * cper, cper)])

            @pl.when((s == 0) & (rem > 0))
            def _():
                pltpu.sync_copy(acc.at[pl.ds(16 * cper, rem)],
                                mir_hbm.at[pl.ds(lo + 16 * cper, rem)])

        @pl.when(pm)
        def _():
            pltpu.sync_copy(accp.at[pl.ds(s * cper, cper)],
                            mip_hbm.at[pl.ds(lo + s * cper, cper)])

            @pl.when((s == 0) & (rem > 0))
            def _():
                pltpu.sync_copy(accp.at[pl.ds(16 * cper, rem)],
                                mip_hbm.at[pl.ds(lo + 16 * cper, rem)])

        plsc.subcore_barrier()
        return 0

    lax.fori_loop(0, 2, one_pass, 0)


def _sc_scatter(msgl, msgr, pmsg, st, n):
    e = st.shape[0]
    half = n // 2
    rows = ((half + 1 + 255) // 256) * 256
    hw = _H // 2
    z = jnp.zeros((rows, hw), F32)
    zp = jnp.zeros((rows, 8), F32)
    body = functools.partial(_scatter_body, e, n)
    f = pl.kernel(
        body,
        out_type=[jax.ShapeDtypeStruct((n, hw), F32),
                  jax.ShapeDtypeStruct((n, hw), F32),
                  jax.ShapeDtypeStruct((n, 8), F32)],
        mesh=_sc_mesh(),
        compiler_params=_SC_PARAMS,
        scratch_types=[
            pltpu.VMEM((80 * _CHUNK,), I32),
            pltpu.VMEM((_CHUNK,), I32),
            pltpu.VMEM((_CHUNK,), I32),
            pltpu.VMEM((_CHUNK, hw), F32),
            pltpu.VMEM((_CHUNK, hw), F32),
            pltpu.VMEM((_CHUNK, 8), F32),
            pltpu.VMEM((_CHUNK, 8), F32),
            pltpu.SemaphoreType.DMA,
            pltpu.SemaphoreType.DMA,
            pltpu.VMEM_SHARED((rows, hw), F32),
            pltpu.VMEM_SHARED((rows, 8), F32),
        ],
    )
    return f(msgl, msgr, pmsg, st, z, zp)


def _node_body(x_ref, mil_ref, mir_ref, pos_ref, mip_ref, w1a_ref,
               w1bl_ref, w1br_ref, b1_ref,
               w2_ref, b2_ref, pw1_ref, pb1_ref, pw2_ref, pb2_ref,
               xn_ref, pn_ref):
    x = x_ref[...]
    h = (jnp.dot(x, w1a_ref[...], preferred_element_type=F32)
         + jnp.dot(mil_ref[...], w1bl_ref[...], preferred_element_type=F32)
         + jnp.dot(mir_ref[...], w1br_ref[...], preferred_element_type=F32)
         + b1_ref[...])
    h = _silu(h)
    xn_ref[...] = x + jnp.dot(h, w2_ref[...], preferred_element_type=F32) + b2_ref[...]
    pos3 = pos_ref[:, 0:3]
    mip3 = mip_ref[:, 0:3]
    hp = pb1_ref[0:1, 0:3]
    for k in range(3):
        hp = hp + pos3[:, k:k + 1] * pw1_ref[k:k + 1, 0:3]
        hp = hp + mip3[:, k:k + 1] * pw1_ref[k + 3:k + 4, 0:3]
    p1 = _silu(hp)
    pd = (p1[:, 0:1] * pw2_ref[0:1, 0:3]
          + p1[:, 1:2] * pw2_ref[1:2, 0:3]
          + p1[:, 2:3] * pw2_ref[2:3, 0:3]) + pb2_ref[0:1, 0:3]
    pn_ref[...] = jnp.concatenate(
        [pos3 + pd, jnp.zeros((pd.shape[0], 5), F32)], axis=1)


def _node_mlp(x, mil, mir, pos8, mip8, w1a, w1bl, w1br, b1, w2, b2,
              pw1, pb1, pw2, pb2, nb=1000):
    n = x.shape[0]
    grid = (n // nb,)
    full = lambda r, c: pl.BlockSpec((r, c), lambda i: (0, 0))
    blk = lambda c: pl.BlockSpec((nb, c), lambda i: (i, 0))
    return pl.pallas_call(
        _node_body,
        grid=grid,
        in_specs=[blk(_H), blk(_H // 2), blk(_H // 2), blk(8), blk(8),
                  full(_H, _H), full(_H // 2, _H), full(_H // 2, _H),
                  full(1, _H), full(_H, _H), full(1, _H),
                  full(8, 8), full(1, 8), full(8, 8), full(1, 8)],
        out_specs=[blk(_H), blk(8)],
        out_shape=[jax.ShapeDtypeStruct((n, _H), F32),
                   jax.ShapeDtypeStruct((n, 8), F32)],
    )(x, mil, mir, pos8, mip8, w1a, w1bl, w1br, b1, w2, b2, pw1, pb1,
      pw2, pb2)


def kernel(x, edge_index, pos,
           fe_w1, fe_b1, fe_w2, fe_b2,
           finf_w, finf_b,
           fh_w1, fh_b1, fh_w2, fh_b2,
           fpos_w1, fpos_b1, fpos_w2, fpos_b2,
           fhpos_w1, fhpos_b1, fhpos_w2, fhpos_b2):
    n = x.shape[0]
    e = edge_index.shape[0]
    e_st = edge_index[:, 0].astype(I32)
    e_end = edge_index[:, 1].astype(I32)

    w1a = fe_w1[:_H]
    w1b = fe_w1[_H:2 * _H]
    w1c = fe_w1[2 * _H:2 * _H + 1]
    b1 = fe_b1.reshape(1, _H)
    wpa = fpos_w1[:_H]
    wpb = fpos_w1[_H:2 * _H]
    wpc = jnp.pad(fpos_w1[2 * _H:2 * _H + 1], ((0, 0), (0, 5)))
    wp = jnp.pad(jnp.concatenate([wpa, wpb], axis=1), ((0, 0), (0, 2)))
    bp = jnp.pad(jnp.concatenate(
        [jnp.zeros((3,), F32), fpos_b1]).reshape(1, 6), ((0, 0), (0, 2)))
    wp2 = jnp.pad(fpos_w2, ((0, 5), (0, 5)))
    bp2 = jnp.pad(fpos_b2.reshape(1, 3), ((0, 0), (0, 5)))
    finf_row = finf_w.reshape(1, _H)
    finfb = finf_b.reshape(1, 1)
    inv_n = jnp.full((1, 1), 1.0 / n, F32)

    xa, xb, pp = _premul(x, w1a, w1b, b1, wp, bp)

    h1pre = _sc_gather_h1pre(xa, xb, e_st, e_end)
    xtra = jnp.concatenate(
        [pos.reshape(-1), pp[:, 0:3].reshape(-1), pp[:, 3:6].reshape(-1)])
    geom = _sc_geom(xtra, e_st, e_end, n).reshape(e, 8)

    msgl, msgr, pmsg = _edge_mlp(h1pre, geom, w1c, fe_w2,
                                 fe_b2.reshape(1, _H),
                                 finf_row, finfb, wpc, wp2, bp2, inv_n)

    mil, mir, mip8 = _sc_scatter(msgl, msgr, pmsg, e_st, n)

    pos8 = jnp.pad(pos, ((0, 0), (0, 5)))
    pw1 = jnp.pad(fhpos_w1, ((0, 2), (0, 5)))
    pb1 = jnp.pad(fhpos_b1.reshape(1, 3), ((0, 0), (0, 5)))
    pw2 = jnp.pad(fhpos_w2, ((0, 5), (0, 5)))
    pb2n = jnp.pad(fhpos_b2.reshape(1, 3), ((0, 0), (0, 5)))
    x_new, pos_new8 = _node_mlp(x, mil, mir, pos8, mip8, fh_w1[:_H],
                                fh_w1[_H:_H + _H // 2],
                                fh_w1[_H + _H // 2:],
                                fh_b1.reshape(1, _H), fh_w2,
                                fh_b2.reshape(1, _H), pw1, pb1, pw2, pb2n)
    return (x_new, edge_index, pos_new8[:, 0:3])
